# baseline (speedup 1.0000x reference)
import functools

import jax
import jax.numpy as jnp
from jax import lax
from jax.experimental import pallas as pl
from jax.experimental.pallas import tpu as pltpu
from jax.experimental.pallas import tpu_sc as plsc

N = 10000
C = 128
OUT = 128
K = 32
E = N * K

NC, NS, L = 2, 16, 16
NW = NC * NS
NPAD = 10240
NPW = NPAD // NW

CE = 640
NCHUNK = E // CE
GB = 16
MS = ((CE + GB - 1) // GB + 1) * GB
NEG = -3.0e38

FB = OUT // L
BN = 1024


def _g_body(xt_ref, w_ref, g_ref):
    xb = xt_ref[...]
    w2 = w_ref[C:, :]
    g_ref[...] = lax.dot_general(
        xb, w2, (((0,), (0,)), ((), ())), preferred_element_type=jnp.float32)


def _run_g(xt, w):
    return pl.pallas_call(
        _g_body,
        grid=(NPAD // BN,),
        in_specs=[
            pl.BlockSpec((C, BN), lambda i: (0, i)),
            pl.BlockSpec((2 * C, OUT), lambda i: (0, 0)),
        ],
        out_specs=pl.BlockSpec((BN, OUT), lambda i: (i, 0)),
        out_shape=jax.ShapeDtypeStruct((NPAD, OUT), jnp.float32),
    )(xt, w)


def _make_sc_segmax():
    mesh = plsc.VectorSubcoreMesh(
        core_axis_name="c", subcore_axis_name="s",
        num_cores=NC, num_subcores=NS)

    @functools.partial(
        pl.kernel,
        out_type=jax.ShapeDtypeStruct((NPAD * OUT,), jnp.float32),
        mesh=mesh,
        scratch_types=[
            pltpu.VMEM((NPW * OUT,), jnp.float32),
            pltpu.VMEM((2, CE), jnp.int32),
            pltpu.VMEM((2, CE), jnp.int32),
            pltpu.VMEM((MS,), jnp.int32),
            pltpu.VMEM((MS,), jnp.int32),
            pltpu.VMEM((2, GB, OUT), jnp.float32),
            pltpu.VMEM_SHARED((N, OUT), jnp.float32),
            pltpu.SemaphoreType.DMA,
            pltpu.SemaphoreType.DMA,
            pltpu.SemaphoreType.DMA,
            pltpu.SemaphoreType.DMA,
        ],
        compiler_params=pltpu.CompilerParams(needs_layout_passes=False),
    )
    def sc_segmax(src_hbm, dst_hbm, g_hbm, m_hbm,
                  m_v, src_v, dst_v, msrc_v, mdst_v, rows_v, gsp,
                  sem_a, sem_b, sem_c, sem_d):
        cid = lax.axis_index("c")
        sid = lax.axis_index("s")
        wid = sid * NC + cid
        base = wid * NPW

        @pl.when(sid < NS - 1)
        def _():
            pltpu.sync_copy(g_hbm.at[pl.ds(sid * 640, 640), :],
                            gsp.at[pl.ds(sid * 640, 640), :])

        @pl.when(sid == NS - 1)
        def _():
            pltpu.sync_copy(g_hbm.at[pl.ds(9600, 400), :],
                            gsp.at[pl.ds(9600, 400), :])
        plsc.subcore_barrier()

        neg = jnp.full((L,), NEG, jnp.float32)
        zero_i = jnp.zeros((L,), jnp.int32)
        lanes = jnp.arange(L, dtype=jnp.int32)
        fidx = [lanes + f * L for f in range(FB)]

        def init_m(r, carry):
            for f in range(FB):
                m_v[pl.ds(r * OUT + f * L, L)] = neg
            return carry
        lax.fori_loop(0, NPW, init_m, 0)

        def init_idx(i, carry):
            msrc_v[pl.ds(i * L, L)] = zero_i
            return carry
        lax.fori_loop(0, MS // L, init_idx, 0)

        def start_chunk(ci, buf, sa):
            pltpu.make_async_copy(
                src_hbm.at[pl.ds(ci * CE, CE)], src_v.at[buf], sa).start()
            pltpu.make_async_copy(
                dst_hbm.at[pl.ds(ci * CE, CE)], dst_v.at[buf], sa).start()

        def wait_chunk(ci, buf, sa):
            pltpu.make_async_copy(
                src_hbm.at[pl.ds(ci * CE, CE)], src_v.at[buf], sa).wait()
            pltpu.make_async_copy(
                dst_hbm.at[pl.ds(ci * CE, CE)], dst_v.at[buf], sa).wait()

        def start_bat(bi, buf, sa):
            pltpu.make_async_copy(
                gsp.at[msrc_v.at[pl.ds(bi * GB, GB)]], rows_v.at[buf], sa
            ).start()

        def wait_bat(bi, buf, sa):
            pltpu.make_async_copy(
                gsp.at[msrc_v.at[pl.ds(bi * GB, GB)]], rows_v.at[buf], sa
            ).wait()

        start_chunk(0, 0, sem_a)

        def chunk_body(ci, carry):
            p = jnp.bitwise_and(ci, 1)
            nxt = ci + 1

            @pl.when(jnp.logical_and(nxt < NCHUNK, p == 1))
            def _():
                start_chunk(nxt, 0, sem_a)

            @pl.when(jnp.logical_and(nxt < NCHUNK, p == 0))
            def _():
                start_chunk(nxt, 1, sem_b)

            @pl.when(p == 0)
            def _():
                wait_chunk(ci, 0, sem_a)

            @pl.when(p == 1)
            def _():
                wait_chunk(ci, 1, sem_b)

            def scan_body(i, cnt):
                d = dst_v[p, pl.ds(i * L, L)]
                dl = d - base
                msk = (dl >= 0) & (dl < NPW)
                s = src_v[p, pl.ds(i * L, L)]
                cum = plsc.cumsum(msk.astype(jnp.int32))
                pos = cnt + cum - 1
                plsc.store_scatter(msrc_v, [pos], s, mask=msk)
                plsc.store_scatter(mdst_v, [pos], dl, mask=msk)
                return cnt + cum[15]

            cnt = lax.fori_loop(0, CE // L, scan_body, jnp.int32(0))
            nbat = (cnt + GB - 1) // GB

            @pl.when(nbat > 0)
            def _():
                start_bat(0, 0, sem_c)

            def bat_body(bi, carry2):
                pb = jnp.bitwise_and(bi, 1)
                nb = bi + 1

                @pl.when(jnp.logical_and(nb < nbat, pb == 1))
                def _():
                    start_bat(nb, 0, sem_c)

                @pl.when(jnp.logical_and(nb < nbat, pb == 0))
                def _():
                    start_bat(nb, 1, sem_d)

                @pl.when(pb == 0)
                def _():
                    wait_bat(bi, 0, sem_c)

                @pl.when(pb == 1)
                def _():
                    wait_bat(bi, 1, sem_d)

                def edge_body(j, carry3):
                    dlo = plsc.load_gather(
                        mdst_v, [jnp.full((L,), bi * GB + j, jnp.int32)])
                    ab = dlo * OUT
                    for f in range(FB):
                        a = ab + fidx[f]
                        m = plsc.load_gather(m_v, [a])
                        r = rows_v[pb, j, pl.ds(f * L, L)]
                        plsc.store_scatter(m_v, [a], jnp.maximum(m, r))
                    return carry3

                nloc = jnp.minimum(cnt - bi * GB, GB)
                lax.fori_loop(0, nloc, edge_body, 0)
                return carry2

            lax.fori_loop(0, nbat, bat_body, 0)
            return carry

        lax.fori_loop(0, NCHUNK, chunk_body, 0)

        pltpu.sync_copy(m_v, m_hbm.at[pl.ds(base * OUT, NPW * OUT)])

    return sc_segmax


_sc_segmax = _make_sc_segmax()


def _ep_body(xt_ref, w_ref, b_ref, m_ref, o_ref):
    xb = xt_ref[...]
    w1m = w_ref[:C, :] - w_ref[C:, :]
    a = lax.dot_general(
        xb, w1m, (((0,), (0,)), ((), ())), preferred_element_type=jnp.float32)
    o_ref[...] = jnp.maximum(a + b_ref[...] + m_ref[...], 0.0)


def _run_epilogue(xt, w, b2, m):
    return pl.pallas_call(
        _ep_body,
        grid=(NPAD // BN,),
        in_specs=[
            pl.BlockSpec((C, BN), lambda i: (0, i)),
            pl.BlockSpec((2 * C, OUT), lambda i: (0, 0)),
            pl.BlockSpec((1, OUT), lambda i: (0, 0)),
            pl.BlockSpec((BN, OUT), lambda i: (i, 0)),
        ],
        out_specs=pl.BlockSpec((BN, OUT), lambda i: (i, 0)),
        out_shape=jax.ShapeDtypeStruct((NPAD, OUT), jnp.float32),
    )(xt, w, b2, m)


def kernel(x, edge_index, W, b):
    xt = x[0, :, :, 0]
    xt = jnp.pad(xt, ((0, 0), (0, NPAD - N)))
    ei = edge_index.reshape(2, E).astype(jnp.int32)
    src = ei[0]
    dst = ei[1]
    w = W.astype(jnp.float32)
    b2 = b.astype(jnp.float32)[None, :]

    g = _run_g(xt, w)
    m = _sc_segmax(src, dst, g[:N]).reshape(NPAD, OUT)
    out = _run_epilogue(xt, w, b2, m)
    return out[:N].T[None, :, :, None]


# speedup vs baseline: 1.0489x; 1.0489x over previous
import functools

import jax
import jax.numpy as jnp
from jax import lax
from jax.experimental import pallas as pl
from jax.experimental.pallas import tpu as pltpu
from jax.experimental.pallas import tpu_sc as plsc

N = 10000
C = 128
OUT = 128
K = 32
E = N * K

NC, NS, L = 2, 16, 16
NW = NC * NS
NPAD = 10240
NPW = NPAD // NW

CE = 640
NCHUNK = E // CE
GB = 16
MS = ((CE + GB - 1) // GB + 1) * GB
NEG = -3.0e38

FB = OUT // L
BN = 1024


def _g_body(xt_ref, w_ref, g_ref):
    xb = xt_ref[...]
    w2 = w_ref[C:, :]
    g_ref[...] = lax.dot_general(
        xb, w2, (((0,), (0,)), ((), ())), preferred_element_type=jnp.float32)


def _run_g(xt, w):
    return pl.pallas_call(
        _g_body,
        grid=(NPAD // BN,),
        in_specs=[
            pl.BlockSpec((C, BN), lambda i: (0, i)),
            pl.BlockSpec((2 * C, OUT), lambda i: (0, 0)),
        ],
        out_specs=pl.BlockSpec((BN, OUT), lambda i: (i, 0)),
        out_shape=jax.ShapeDtypeStruct((NPAD, OUT), jnp.float32),
    )(xt, w)


def _make_sc_segmax():
    mesh = plsc.VectorSubcoreMesh(
        core_axis_name="c", subcore_axis_name="s",
        num_cores=NC, num_subcores=NS)

    @functools.partial(
        pl.kernel,
        out_type=jax.ShapeDtypeStruct((NPAD * OUT,), jnp.float32),
        mesh=mesh,
        scratch_types=[
            pltpu.VMEM((NPW * OUT,), jnp.float32),
            pltpu.VMEM((2, CE), jnp.int32),
            pltpu.VMEM((2, CE), jnp.int32),
            pltpu.VMEM((MS,), jnp.int32),
            pltpu.VMEM((MS,), jnp.int32),
            pltpu.VMEM((2, GB, OUT), jnp.float32),
            pltpu.VMEM_SHARED((N, OUT), jnp.float32),
            pltpu.SemaphoreType.DMA,
            pltpu.SemaphoreType.DMA,
            pltpu.SemaphoreType.DMA,
            pltpu.SemaphoreType.DMA,
        ],
        compiler_params=pltpu.CompilerParams(needs_layout_passes=False),
    )
    def sc_segmax(src_hbm, dst_hbm, g_hbm, m_hbm,
                  m_v, src_v, dst_v, msrc_v, mdst_v, rows_v, gsp,
                  sem_a, sem_b, sem_c, sem_d):
        cid = lax.axis_index("c")
        sid = lax.axis_index("s")
        wid = sid * NC + cid
        base = wid * NPW

        @pl.when(sid < NS - 1)
        def _():
            pltpu.sync_copy(g_hbm.at[pl.ds(sid * 640, 640), :],
                            gsp.at[pl.ds(sid * 640, 640), :])

        @pl.when(sid == NS - 1)
        def _():
            pltpu.sync_copy(g_hbm.at[pl.ds(9600, 400), :],
                            gsp.at[pl.ds(9600, 400), :])
        plsc.subcore_barrier()

        neg = jnp.full((L,), NEG, jnp.float32)
        zero_i = jnp.zeros((L,), jnp.int32)
        lanes = jnp.arange(L, dtype=jnp.int32)
        fidx = [lanes + f * L for f in range(FB)]

        def init_m(r, carry):
            for f in range(FB):
                m_v[pl.ds(r * OUT + f * L, L)] = neg
            return carry
        lax.fori_loop(0, NPW, init_m, 0)

        def init_idx(i, carry):
            msrc_v[pl.ds(i * L, L)] = zero_i
            mdst_v[pl.ds(i * L, L)] = zero_i
            return carry
        lax.fori_loop(0, MS // L, init_idx, 0)

        def start_chunk(ci, buf, sa):
            pltpu.make_async_copy(
                src_hbm.at[pl.ds(ci * CE, CE)], src_v.at[buf], sa).start()
            pltpu.make_async_copy(
                dst_hbm.at[pl.ds(ci * CE, CE)], dst_v.at[buf], sa).start()

        def wait_chunk(ci, buf, sa):
            pltpu.make_async_copy(
                src_hbm.at[pl.ds(ci * CE, CE)], src_v.at[buf], sa).wait()
            pltpu.make_async_copy(
                dst_hbm.at[pl.ds(ci * CE, CE)], dst_v.at[buf], sa).wait()

        def start_bat(bi, buf, sa):
            pltpu.make_async_copy(
                gsp.at[msrc_v.at[pl.ds(bi * GB, GB)]], rows_v.at[buf], sa
            ).start()

        def wait_bat(bi, buf, sa):
            pltpu.make_async_copy(
                gsp.at[msrc_v.at[pl.ds(bi * GB, GB)]], rows_v.at[buf], sa
            ).wait()

        start_chunk(0, 0, sem_a)

        def chunk_body(ci, carry):
            p = jnp.bitwise_and(ci, 1)
            nxt = ci + 1

            @pl.when(jnp.logical_and(nxt < NCHUNK, p == 1))
            def _():
                start_chunk(nxt, 0, sem_a)

            @pl.when(jnp.logical_and(nxt < NCHUNK, p == 0))
            def _():
                start_chunk(nxt, 1, sem_b)

            @pl.when(p == 0)
            def _():
                wait_chunk(ci, 0, sem_a)

            @pl.when(p == 1)
            def _():
                wait_chunk(ci, 1, sem_b)

            def scan_body(i, cnt):
                d0 = dst_v[p, pl.ds((2 * i) * L, L)]
                d1 = dst_v[p, pl.ds((2 * i + 1) * L, L)]
                dl0 = d0 - base
                dl1 = d1 - base
                mk0 = (dl0 >= 0) & (dl0 < NPW)
                mk1 = (dl1 >= 0) & (dl1 < NPW)
                s0 = src_v[p, pl.ds((2 * i) * L, L)]
                s1 = src_v[p, pl.ds((2 * i + 1) * L, L)]
                c0 = plsc.cumsum(mk0.astype(jnp.int32))
                c1 = plsc.cumsum(mk1.astype(jnp.int32))
                pos0 = cnt + c0 - 1
                t0 = c0[15]
                pos1 = (cnt + t0) + c1 - 1
                plsc.store_scatter(msrc_v, [pos0], s0, mask=mk0)
                plsc.store_scatter(mdst_v, [pos0], dl0, mask=mk0)
                plsc.store_scatter(msrc_v, [pos1], s1, mask=mk1)
                plsc.store_scatter(mdst_v, [pos1], dl1, mask=mk1)
                return cnt + t0 + c1[15]

            cnt = lax.fori_loop(0, CE // (2 * L), scan_body, jnp.int32(0))
            nbat = (cnt + GB - 1) // GB

            @pl.when(nbat > 0)
            def _():
                start_bat(0, 0, sem_c)

            def bat_body(bi, carry2):
                pb = jnp.bitwise_and(bi, 1)
                nb = bi + 1

                @pl.when(jnp.logical_and(nb < nbat, pb == 1))
                def _():
                    start_bat(nb, 0, sem_c)

                @pl.when(jnp.logical_and(nb < nbat, pb == 0))
                def _():
                    start_bat(nb, 1, sem_d)

                @pl.when(pb == 0)
                def _():
                    wait_bat(bi, 0, sem_c)

                @pl.when(pb == 1)
                def _():
                    wait_bat(bi, 1, sem_d)

                nloc = jnp.minimum(cnt - bi * GB, GB)

                def edge_body(j2, carry3):
                    j0 = 2 * j2
                    j1 = 2 * j2 + 1
                    e0 = bi * GB + j0
                    dlo0 = plsc.load_gather(
                        mdst_v, [jnp.full((L,), e0, jnp.int32)])
                    dlo1 = plsc.load_gather(
                        mdst_v, [jnp.full((L,), e0 + 1, jnp.int32)])
                    ok1 = jnp.full((L,), j1 < nloc)
                    ab0 = dlo0 * OUT
                    ab1 = dlo1 * OUT
                    for f in range(FB):
                        a0 = ab0 + fidx[f]
                        a1 = ab1 + fidx[f]
                        m0 = plsc.load_gather(m_v, [a0])
                        r0 = rows_v[pb, j0, pl.ds(f * L, L)]
                        plsc.store_scatter(m_v, [a0], jnp.maximum(m0, r0))
                        m1 = plsc.load_gather(m_v, [a1])
                        r1 = rows_v[pb, j1, pl.ds(f * L, L)]
                        plsc.store_scatter(
                            m_v, [a1], jnp.maximum(m1, r1), mask=ok1)
                    return carry3

                lax.fori_loop(0, (nloc + 1) // 2, edge_body, 0)
                return carry2

            lax.fori_loop(0, nbat, bat_body, 0)
            return carry

        lax.fori_loop(0, NCHUNK, chunk_body, 0)

        pltpu.sync_copy(m_v, m_hbm.at[pl.ds(base * OUT, NPW * OUT)])

    return sc_segmax


_sc_segmax = _make_sc_segmax()


def _ep_body(xt_ref, w_ref, b_ref, m_ref, o_ref):
    xb = xt_ref[...]
    w1m = w_ref[:C, :] - w_ref[C:, :]
    a = lax.dot_general(
        xb, w1m, (((0,), (0,)), ((), ())), preferred_element_type=jnp.float32)
    o_ref[...] = jnp.maximum(a + b_ref[...] + m_ref[...], 0.0)


def _run_epilogue(xt, w, b2, m):
    return pl.pallas_call(
        _ep_body,
        grid=(NPAD // BN,),
        in_specs=[
            pl.BlockSpec((C, BN), lambda i: (0, i)),
            pl.BlockSpec((2 * C, OUT), lambda i: (0, 0)),
            pl.BlockSpec((1, OUT), lambda i: (0, 0)),
            pl.BlockSpec((BN, OUT), lambda i: (i, 0)),
        ],
        out_specs=pl.BlockSpec((BN, OUT), lambda i: (i, 0)),
        out_shape=jax.ShapeDtypeStruct((NPAD, OUT), jnp.float32),
    )(xt, w, b2, m)


def kernel(x, edge_index, W, b):
    xt = x[0, :, :, 0]
    xt = jnp.pad(xt, ((0, 0), (0, NPAD - N)))
    ei = edge_index.reshape(2, E).astype(jnp.int32)
    src = ei[0]
    dst = ei[1]
    w = W.astype(jnp.float32)
    b2 = b.astype(jnp.float32)[None, :]

    g = _run_g(xt, w)
    m = _sc_segmax(src, dst, g[:N]).reshape(NPAD, OUT)
    out = _run_epilogue(xt, w, b2, m)
    return out[:N].T[None, :, :, None]


# speedup vs baseline: 1.0698x; 1.0200x over previous
import functools

import jax
import jax.numpy as jnp
from jax import lax
from jax.experimental import pallas as pl
from jax.experimental.pallas import tpu as pltpu
from jax.experimental.pallas import tpu_sc as plsc

N = 10000
C = 128
OUT = 128
K = 32
E = N * K

NC, NS, L = 2, 16, 16
NW = NC * NS
NPAD = 10240
NPW = NPAD // NW

CE = 768
NCHUNK = -(-E // CE)
EPAD = NCHUNK * CE
GB = 16
MS = ((CE + GB - 1) // GB + 1) * GB
NEG = -3.0e38

FB = OUT // L
BN = 1024


def _g_body(xt_ref, w_ref, g_ref):
    xb = xt_ref[...]
    w2 = w_ref[C:, :]
    g_ref[...] = lax.dot_general(
        xb, w2, (((0,), (0,)), ((), ())), preferred_element_type=jnp.float32)


def _run_g(xt, w):
    return pl.pallas_call(
        _g_body,
        grid=(NPAD // BN,),
        in_specs=[
            pl.BlockSpec((C, BN), lambda i: (0, i)),
            pl.BlockSpec((2 * C, OUT), lambda i: (0, 0)),
        ],
        out_specs=pl.BlockSpec((BN, OUT), lambda i: (i, 0)),
        out_shape=jax.ShapeDtypeStruct((NPAD, OUT), jnp.float32),
    )(xt, w)


def _make_sc_segmax():
    mesh = plsc.VectorSubcoreMesh(
        core_axis_name="c", subcore_axis_name="s",
        num_cores=NC, num_subcores=NS)

    @functools.partial(
        pl.kernel,
        out_type=jax.ShapeDtypeStruct((NPAD * OUT,), jnp.float32),
        mesh=mesh,
        scratch_types=[
            pltpu.VMEM((NPW * OUT,), jnp.float32),
            pltpu.VMEM((2, CE), jnp.int32),
            pltpu.VMEM((2, CE), jnp.int32),
            pltpu.VMEM((MS,), jnp.int32),
            pltpu.VMEM((MS,), jnp.int32),
            pltpu.VMEM((2, GB, OUT), jnp.float32),
            pltpu.VMEM_SHARED((N, OUT), jnp.float32),
            pltpu.SemaphoreType.DMA,
            pltpu.SemaphoreType.DMA,
            pltpu.SemaphoreType.DMA,
            pltpu.SemaphoreType.DMA,
        ],
        compiler_params=pltpu.CompilerParams(needs_layout_passes=False),
    )
    def sc_segmax(src_hbm, dst_hbm, g_hbm, m_hbm,
                  m_v, src_v, dst_v, msrc_v, mdst_v, rows_v, gsp,
                  sem_a, sem_b, sem_c, sem_d):
        cid = lax.axis_index("c")
        sid = lax.axis_index("s")
        wid = sid * NC + cid
        base = wid * NPW

        @pl.when(sid < NS - 1)
        def _():
            pltpu.sync_copy(g_hbm.at[pl.ds(sid * 640, 640), :],
                            gsp.at[pl.ds(sid * 640, 640), :])

        @pl.when(sid == NS - 1)
        def _():
            pltpu.sync_copy(g_hbm.at[pl.ds(9600, 400), :],
                            gsp.at[pl.ds(9600, 400), :])
        plsc.subcore_barrier()

        neg = jnp.full((L,), NEG, jnp.float32)
        zero_i = jnp.zeros((L,), jnp.int32)
        lanes = jnp.arange(L, dtype=jnp.int32)
        fidx = [lanes + f * L for f in range(FB)]

        def init_m(r, carry):
            for f in range(FB):
                m_v[pl.ds(r * OUT + f * L, L)] = neg
            return carry
        lax.fori_loop(0, NPW, init_m, 0)

        def init_idx(i, carry):
            msrc_v[pl.ds(i * L, L)] = zero_i
            mdst_v[pl.ds(i * L, L)] = zero_i
            return carry
        lax.fori_loop(0, MS // L, init_idx, 0)

        def start_chunk(ci, buf, sa):
            pltpu.make_async_copy(
                src_hbm.at[pl.ds(ci * CE, CE)], src_v.at[buf], sa).start()
            pltpu.make_async_copy(
                dst_hbm.at[pl.ds(ci * CE, CE)], dst_v.at[buf], sa).start()

        def wait_chunk(ci, buf, sa):
            pltpu.make_async_copy(
                src_hbm.at[pl.ds(ci * CE, CE)], src_v.at[buf], sa).wait()
            pltpu.make_async_copy(
                dst_hbm.at[pl.ds(ci * CE, CE)], dst_v.at[buf], sa).wait()

        def start_bat(bi, buf, sa):
            pltpu.make_async_copy(
                gsp.at[msrc_v.at[pl.ds(bi * GB, GB)]], rows_v.at[buf], sa
            ).start()

        def wait_bat(bi, buf, sa):
            pltpu.make_async_copy(
                gsp.at[msrc_v.at[pl.ds(bi * GB, GB)]], rows_v.at[buf], sa
            ).wait()

        start_chunk(0, 0, sem_a)

        def chunk_body(ci, carry):
            p = jnp.bitwise_and(ci, 1)
            nxt = ci + 1

            @pl.when(jnp.logical_and(nxt < NCHUNK, p == 1))
            def _():
                start_chunk(nxt, 0, sem_a)

            @pl.when(jnp.logical_and(nxt < NCHUNK, p == 0))
            def _():
                start_chunk(nxt, 1, sem_b)

            @pl.when(p == 0)
            def _():
                wait_chunk(ci, 0, sem_a)

            @pl.when(p == 1)
            def _():
                wait_chunk(ci, 1, sem_b)

            def scan_body(i, cnt):
                d0 = dst_v[p, pl.ds((2 * i) * L, L)]
                d1 = dst_v[p, pl.ds((2 * i + 1) * L, L)]
                dl0 = d0 - base
                dl1 = d1 - base
                mk0 = (dl0 >= 0) & (dl0 < NPW)
                mk1 = (dl1 >= 0) & (dl1 < NPW)
                s0 = src_v[p, pl.ds((2 * i) * L, L)]
                s1 = src_v[p, pl.ds((2 * i + 1) * L, L)]
                c0 = plsc.cumsum(mk0.astype(jnp.int32))
                c1 = plsc.cumsum(mk1.astype(jnp.int32))
                pos0 = cnt + c0 - 1
                t0 = c0[15]
                pos1 = (cnt + t0) + c1 - 1
                plsc.store_scatter(msrc_v, [pos0], s0, mask=mk0)
                plsc.store_scatter(mdst_v, [pos0], dl0, mask=mk0)
                plsc.store_scatter(msrc_v, [pos1], s1, mask=mk1)
                plsc.store_scatter(mdst_v, [pos1], dl1, mask=mk1)
                return cnt + t0 + c1[15]

            cnt = lax.fori_loop(0, CE // (2 * L), scan_body, jnp.int32(0))
            nbat = (cnt + GB - 1) // GB

            @pl.when(nbat > 0)
            def _():
                start_bat(0, 0, sem_c)

            def bat_body(bi, carry2):
                pb = jnp.bitwise_and(bi, 1)
                nb = bi + 1

                @pl.when(jnp.logical_and(nb < nbat, pb == 1))
                def _():
                    start_bat(nb, 0, sem_c)

                @pl.when(jnp.logical_and(nb < nbat, pb == 0))
                def _():
                    start_bat(nb, 1, sem_d)

                @pl.when(pb == 0)
                def _():
                    wait_bat(bi, 0, sem_c)

                @pl.when(pb == 1)
                def _():
                    wait_bat(bi, 1, sem_d)

                nloc = jnp.minimum(cnt - bi * GB, GB)

                def edge_body(j2, carry3):
                    j0 = 2 * j2
                    j1 = 2 * j2 + 1
                    e0 = bi * GB + j0
                    dlo0 = plsc.load_gather(
                        mdst_v, [jnp.full((L,), e0, jnp.int32)])
                    dlo1 = plsc.load_gather(
                        mdst_v, [jnp.full((L,), e0 + 1, jnp.int32)])
                    ok1 = jnp.full((L,), j1 < nloc)
                    ab0 = dlo0 * OUT
                    ab1 = dlo1 * OUT
                    for f in range(FB):
                        a0 = ab0 + fidx[f]
                        a1 = ab1 + fidx[f]
                        m0 = plsc.load_gather(m_v, [a0])
                        r0 = rows_v[pb, j0, pl.ds(f * L, L)]
                        plsc.store_scatter(m_v, [a0], jnp.maximum(m0, r0))
                        m1 = plsc.load_gather(m_v, [a1])
                        r1 = rows_v[pb, j1, pl.ds(f * L, L)]
                        plsc.store_scatter(
                            m_v, [a1], jnp.maximum(m1, r1), mask=ok1)
                    return carry3

                lax.fori_loop(0, (nloc + 1) // 2, edge_body, 0)
                return carry2

            lax.fori_loop(0, nbat, bat_body, 0)
            return carry

        lax.fori_loop(0, NCHUNK, chunk_body, 0)

        pltpu.sync_copy(m_v, m_hbm.at[pl.ds(base * OUT, NPW * OUT)])

    return sc_segmax


_sc_segmax = _make_sc_segmax()


def _ep_body(xt_ref, w_ref, b_ref, m_ref, o_ref):
    xb = xt_ref[...]
    w1m = w_ref[:C, :] - w_ref[C:, :]
    a = lax.dot_general(
        xb, w1m, (((0,), (0,)), ((), ())), preferred_element_type=jnp.float32)
    o_ref[...] = jnp.maximum(a + b_ref[...] + m_ref[...], 0.0)


def _run_epilogue(xt, w, b2, m):
    return pl.pallas_call(
        _ep_body,
        grid=(NPAD // BN,),
        in_specs=[
            pl.BlockSpec((C, BN), lambda i: (0, i)),
            pl.BlockSpec((2 * C, OUT), lambda i: (0, 0)),
            pl.BlockSpec((1, OUT), lambda i: (0, 0)),
            pl.BlockSpec((BN, OUT), lambda i: (i, 0)),
        ],
        out_specs=pl.BlockSpec((BN, OUT), lambda i: (i, 0)),
        out_shape=jax.ShapeDtypeStruct((NPAD, OUT), jnp.float32),
    )(xt, w, b2, m)


def kernel(x, edge_index, W, b):
    xt = x[0, :, :, 0]
    xt = jnp.pad(xt, ((0, 0), (0, NPAD - N)))
    ei = edge_index.reshape(2, E).astype(jnp.int32)
    src = jnp.pad(ei[0], (0, EPAD - E))
    dst = jnp.pad(ei[1], (0, EPAD - E), constant_values=jnp.int32(1 << 30))
    w = W.astype(jnp.float32)
    b2 = b.astype(jnp.float32)[None, :]

    g = _run_g(xt, w)
    m = _sc_segmax(src, dst, g[:N]).reshape(NPAD, OUT)
    out = _run_epilogue(xt, w, b2, m)
    return out[:N].T[None, :, :, None]


# speedup vs baseline: 1.2117x; 1.1326x over previous
import functools

import jax
import jax.numpy as jnp
from jax import lax
from jax.experimental import pallas as pl
from jax.experimental.pallas import tpu as pltpu
from jax.experimental.pallas import tpu_sc as plsc

N = 10000
C = 128
OUT = 128
K = 32
E = N * K

NC, NS, L = 2, 16, 16
NW = NC * NS
NPAD = 10240
NPW = NPAD // NW

CE = 768
NCHUNK = -(-E // CE)
EPAD = NCHUNK * CE
GB = 16
MS = ((CE + GB - 1) // GB + 1) * GB
NEG = -3.0e38

FB = OUT // L
BN = 1024


def _g_body(xt_ref, w_ref, g_ref):
    xb = xt_ref[...]
    w2 = w_ref[C:, :]
    g_ref[...] = lax.dot_general(
        xb, w2, (((0,), (0,)), ((), ())), preferred_element_type=jnp.float32)


def _run_g(xt, w):
    return pl.pallas_call(
        _g_body,
        grid=(NPAD // BN,),
        in_specs=[
            pl.BlockSpec((C, BN), lambda i: (0, i)),
            pl.BlockSpec((2 * C, OUT), lambda i: (0, 0)),
        ],
        out_specs=pl.BlockSpec((BN, OUT), lambda i: (i, 0)),
        out_shape=jax.ShapeDtypeStruct((NPAD, OUT), jnp.float32),
    )(xt, w)


def _make_sc_segmax():
    mesh = plsc.VectorSubcoreMesh(
        core_axis_name="c", subcore_axis_name="s",
        num_cores=NC, num_subcores=NS)

    @functools.partial(
        pl.kernel,
        out_type=jax.ShapeDtypeStruct((NPAD * OUT,), jnp.float32),
        mesh=mesh,
        scratch_types=[
            pltpu.VMEM((NPW * OUT,), jnp.float32),
            pltpu.VMEM((2, CE), jnp.int32),
            pltpu.VMEM((2, CE), jnp.int32),
            pltpu.VMEM((MS,), jnp.int32),
            pltpu.VMEM((MS,), jnp.int32),
            pltpu.VMEM((2, GB, OUT), jnp.float32),
            pltpu.VMEM_SHARED((N, OUT), jnp.float32),
            pltpu.SemaphoreType.DMA,
            pltpu.SemaphoreType.DMA,
            pltpu.SemaphoreType.DMA,
            pltpu.SemaphoreType.DMA,
        ],
        compiler_params=pltpu.CompilerParams(needs_layout_passes=False),
    )
    def sc_segmax(src_hbm, dst_hbm, g_hbm, m_hbm,
                  m_v, src_v, dst_v, msrc_v, mdst_v, rows_v, gsp,
                  sem_a, sem_b, sem_c, sem_d):
        cid = lax.axis_index("c")
        sid = lax.axis_index("s")
        wid = sid * NC + cid
        base = wid * NPW

        @pl.when(sid < NS - 1)
        def _():
            pltpu.sync_copy(g_hbm.at[pl.ds(sid * 640, 640), :],
                            gsp.at[pl.ds(sid * 640, 640), :])

        @pl.when(sid == NS - 1)
        def _():
            pltpu.sync_copy(g_hbm.at[pl.ds(9600, 400), :],
                            gsp.at[pl.ds(9600, 400), :])
        plsc.subcore_barrier()

        neg = jnp.full((L,), NEG, jnp.float32)
        zero_i = jnp.zeros((L,), jnp.int32)
        lanes = jnp.arange(L, dtype=jnp.int32)
        fidx = [lanes + f * L for f in range(FB)]

        def init_m(r, carry):
            for f in range(FB):
                m_v[pl.ds(r * OUT + f * L, L)] = neg
            return carry
        lax.fori_loop(0, NPW, init_m, 0)

        def init_idx(i, carry):
            msrc_v[pl.ds(i * L, L)] = zero_i
            mdst_v[pl.ds(i * L, L)] = zero_i
            return carry
        lax.fori_loop(0, MS // L, init_idx, 0)

        def start_chunk(ci, buf, sa):
            pltpu.make_async_copy(
                src_hbm.at[pl.ds(ci * CE, CE)], src_v.at[buf], sa).start()
            pltpu.make_async_copy(
                dst_hbm.at[pl.ds(ci * CE, CE)], dst_v.at[buf], sa).start()

        def wait_chunk(ci, buf, sa):
            pltpu.make_async_copy(
                src_hbm.at[pl.ds(ci * CE, CE)], src_v.at[buf], sa).wait()
            pltpu.make_async_copy(
                dst_hbm.at[pl.ds(ci * CE, CE)], dst_v.at[buf], sa).wait()

        def start_bat(bi, buf, sa):
            pltpu.make_async_copy(
                gsp.at[msrc_v.at[pl.ds(bi * GB, GB)]], rows_v.at[buf], sa
            ).start()

        def wait_bat(bi, buf, sa):
            pltpu.make_async_copy(
                gsp.at[msrc_v.at[pl.ds(bi * GB, GB)]], rows_v.at[buf], sa
            ).wait()

        start_chunk(0, 0, sem_a)

        def chunk_body(ci, carry):
            p = jnp.bitwise_and(ci, 1)
            nxt = ci + 1

            @pl.when(jnp.logical_and(nxt < NCHUNK, p == 1))
            def _():
                start_chunk(nxt, 0, sem_a)

            @pl.when(jnp.logical_and(nxt < NCHUNK, p == 0))
            def _():
                start_chunk(nxt, 1, sem_b)

            @pl.when(p == 0)
            def _():
                wait_chunk(ci, 0, sem_a)

            @pl.when(p == 1)
            def _():
                wait_chunk(ci, 1, sem_b)

            def scan_body(i, cnt):
                ds_ = [dst_v[p, pl.ds((4 * i + k) * L, L)] for k in range(4)]
                ss = [src_v[p, pl.ds((4 * i + k) * L, L)] for k in range(4)]
                dls = [d - base for d in ds_]
                mks = [(dl >= 0) & (dl < NPW) for dl in dls]
                cs = [plsc.cumsum(mk.astype(jnp.int32)) for mk in mks]
                run = cnt
                for k in range(4):
                    pos = run + cs[k] - 1
                    plsc.store_scatter(msrc_v, [pos], ss[k], mask=mks[k])
                    plsc.store_scatter(mdst_v, [pos], dls[k], mask=mks[k])
                    run = run + cs[k][15]
                return run

            cnt = lax.fori_loop(0, CE // (4 * L), scan_body, jnp.int32(0))
            nbat = (cnt + GB - 1) // GB

            @pl.when(nbat > 0)
            def _():
                start_bat(0, 0, sem_c)

            def bat_body(bi, carry2):
                pb = jnp.bitwise_and(bi, 1)
                nb = bi + 1

                @pl.when(jnp.logical_and(nb < nbat, pb == 1))
                def _():
                    start_bat(nb, 0, sem_c)

                @pl.when(jnp.logical_and(nb < nbat, pb == 0))
                def _():
                    start_bat(nb, 1, sem_d)

                @pl.when(pb == 0)
                def _():
                    wait_bat(bi, 0, sem_c)

                @pl.when(pb == 1)
                def _():
                    wait_bat(bi, 1, sem_d)

                nloc = jnp.minimum(cnt - bi * GB, GB)

                def edge_body(j2, carry3):
                    j0 = 2 * j2
                    j1 = 2 * j2 + 1
                    e0 = bi * GB + j0
                    dlo0 = plsc.load_gather(
                        mdst_v, [jnp.full((L,), e0, jnp.int32)])
                    dlo1 = plsc.load_gather(
                        mdst_v, [jnp.full((L,), e0 + 1, jnp.int32)])
                    ok1 = jnp.full((L,), j1 < nloc)
                    ab0 = dlo0 * OUT
                    ab1 = dlo1 * OUT
                    for f in range(FB):
                        a0 = ab0 + fidx[f]
                        a1 = ab1 + fidx[f]
                        m0 = plsc.load_gather(m_v, [a0])
                        r0 = rows_v[pb, j0, pl.ds(f * L, L)]
                        plsc.store_scatter(m_v, [a0], jnp.maximum(m0, r0))
                        m1 = plsc.load_gather(m_v, [a1])
                        r1 = rows_v[pb, j1, pl.ds(f * L, L)]
                        plsc.store_scatter(
                            m_v, [a1], jnp.maximum(m1, r1), mask=ok1)
                    return carry3

                lax.fori_loop(0, (nloc + 1) // 2, edge_body, 0)
                return carry2

            lax.fori_loop(0, nbat, bat_body, 0)
            return carry

        lax.fori_loop(0, NCHUNK, chunk_body, 0)

        pltpu.sync_copy(m_v, m_hbm.at[pl.ds(base * OUT, NPW * OUT)])

    return sc_segmax


_sc_segmax = _make_sc_segmax()


def _ep_body(xt_ref, w_ref, b_ref, m_ref, o_ref):
    xb = xt_ref[...]
    w1m = w_ref[:C, :] - w_ref[C:, :]
    a = lax.dot_general(
        xb, w1m, (((0,), (0,)), ((), ())), preferred_element_type=jnp.float32)
    o_ref[...] = jnp.maximum(a + b_ref[...] + m_ref[...], 0.0)


def _run_epilogue(xt, w, b2, m):
    return pl.pallas_call(
        _ep_body,
        grid=(NPAD // BN,),
        in_specs=[
            pl.BlockSpec((C, BN), lambda i: (0, i)),
            pl.BlockSpec((2 * C, OUT), lambda i: (0, 0)),
            pl.BlockSpec((1, OUT), lambda i: (0, 0)),
            pl.BlockSpec((BN, OUT), lambda i: (i, 0)),
        ],
        out_specs=pl.BlockSpec((BN, OUT), lambda i: (i, 0)),
        out_shape=jax.ShapeDtypeStruct((NPAD, OUT), jnp.float32),
    )(xt, w, b2, m)


def kernel(x, edge_index, W, b):
    xt = x[0, :, :, 0]
    xt = jnp.pad(xt, ((0, 0), (0, NPAD - N)))
    ei = edge_index.reshape(2, E).astype(jnp.int32)
    src = jnp.pad(ei[0], (0, EPAD - E))
    dst = jnp.pad(ei[1], (0, EPAD - E), constant_values=jnp.int32(1 << 30))
    w = W.astype(jnp.float32)
    b2 = b.astype(jnp.float32)[None, :]

    g = _run_g(xt, w)
    m = _sc_segmax(src, dst, g[:N]).reshape(NPAD, OUT)
    out = _run_epilogue(xt, w, b2, m)
    return out[:N].T[None, :, :, None]
